# baseline (speedup 1.0000x reference)
import jax
import jax.numpy as jnp
from jax import lax
from jax.experimental import pallas as pl
from jax.experimental.pallas import tpu as pltpu, tpu_sc as plsc

N = 10000
D = 128
E = 320000
GAMMA = 0.5

NC = 2
NS = 16
L = 16

EPC = E // NC
EPT = EPC // NS
CH = 40
NCHUNK = EPT // CH
UN = 8
NP = (NCHUNK - 2) // UN
RCH = 80
NRCH = N // RCH
RITER = -(-NRCH // NS)


def _body(h_hbm, src_hbm, dst_hbm, e_hbm, out0, out1,
          srcv, dstv, ev, grow, srow, fidx_v, fbuf_v, acc,
          gsem, ssem, lsem):
    c = lax.axis_index("c")
    s = lax.axis_index("s")
    lane = lax.iota(jnp.int32, L)
    ebase = c * EPC + s * EPT

    def init_chunk(i, _):
        cid = s + i * NS

        @pl.when(cid < NRCH)
        def _():
            base_r = cid * RCH

            @pl.when(c == 0)
            def _():
                for v in range(RCH // L):
                    fidx_v[pl.ds(v * L, L)] = base_r + v * L + lane
                pltpu.async_copy(h_hbm.at[fidx_v], fbuf_v,
                                 gsem.at[0]).wait()

                @plsc.parallel_loop(0, RCH, unroll=2)
                def _(j):
                    for q in range(D // L):
                        sl = pl.ds(q * L, L)
                        fbuf_v[j, sl] = fbuf_v[j, sl] * (-GAMMA)

            @pl.when(c == 1)
            def _():
                zero = jnp.zeros((L,), jnp.float32)

                @plsc.parallel_loop(0, RCH, unroll=2)
                def _(j):
                    for q in range(D // L):
                        fbuf_v[j, pl.ds(q * L, L)] = zero
            pltpu.sync_copy(fbuf_v, acc.at[pl.ds(base_r, RCH)])
        return 0
    lax.fori_loop(0, RITER, init_chunk, 0)
    plsc.subcore_barrier()

    def issue_load(i, k):
        off = ebase + i * CH
        pltpu.async_copy(src_hbm.at[pl.ds(off, CH)], srcv.at[k], lsem.at[k])
        pltpu.async_copy(dst_hbm.at[pl.ds(off, CH)], dstv.at[k], lsem.at[k])
        pltpu.async_copy(e_hbm.at[pl.ds(off, CH)], ev.at[k], lsem.at[k])

    def wait_load(i, k):
        off = ebase + i * CH
        pltpu.make_async_copy(src_hbm.at[pl.ds(off, CH)], srcv.at[k],
                              lsem.at[k]).wait()
        pltpu.make_async_copy(dst_hbm.at[pl.ds(off, CH)], dstv.at[k],
                              lsem.at[k]).wait()
        pltpu.make_async_copy(e_hbm.at[pl.ds(off, CH)], ev.at[k],
                              lsem.at[k]).wait()

    def issue_gather(k, b):
        pltpu.async_copy(h_hbm.at[srcv.at[k]], grow.at[b], gsem.at[b])

    def wait_gather(k, b):
        pltpu.make_async_copy(h_hbm.at[srcv.at[k]], grow.at[b],
                              gsem.at[b]).wait()

    def issue_scatter(k, b):
        pltpu.async_copy(srow.at[b], acc.at[dstv.at[k]], ssem.at[b],
                         add=True)

    def wait_scatter(k, b):
        pltpu.make_async_copy(srow.at[b], acc.at[dstv.at[k]],
                              ssem.at[b]).wait()

    def mul_chunk(k, b):
        @plsc.parallel_loop(0, CH, unroll=4)
        def _(j):
            e16 = ev[k, pl.ds(j, L)]
            eb = lax.gather(
                e16, jnp.zeros((L, 1), jnp.int32),
                lax.GatherDimensionNumbers(
                    offset_dims=(), collapsed_slice_dims=(0,),
                    start_index_map=(0,)),
                (1,), mode=lax.GatherScatterMode.PROMISE_IN_BOUNDS)
            for q in range(D // L):
                sl = pl.ds(q * L, L)
                srow[b, j, sl] = grow[b, j, sl] * eb

    def chunk_body(i, k):
        b = k % 2
        wait_gather(k, b)

        @pl.when(i >= 2)
        def _():
            wait_scatter((k + 2) % UN, b)
        mul_chunk(k, b)
        issue_scatter(k, b)

        @pl.when(i + 2 < NCHUNK)
        def _():
            wait_load(i + 2, (k + 2) % UN)
            issue_gather((k + 2) % UN, b)

        @pl.when(i + 4 < NCHUNK)
        def _():
            issue_load(i + 4, (k + 4) % UN)

    issue_load(0, 0)
    issue_load(1, 1)
    issue_load(2, 2)
    issue_load(3, 3)
    wait_load(0, 0)
    issue_gather(0, 0)
    wait_load(1, 1)
    issue_gather(1, 1)

    def pipe_step(p, _):
        for k in range(UN):
            chunk_body(p * UN + k, k)
        return 0
    lax.fori_loop(0, NP, pipe_step, 0)
    chunk_body(NCHUNK - 2, (NCHUNK - 2) % UN)
    chunk_body(NCHUNK - 1, (NCHUNK - 1) % UN)
    wait_scatter((NCHUNK - 2) % UN, (NCHUNK - 2) % 2)
    wait_scatter((NCHUNK - 1) % UN, (NCHUNK - 1) % 2)
    plsc.subcore_barrier()

    def out_chunk(i, _):
        cid = s + i * NS

        @pl.when(cid < NRCH)
        def _():
            base_r = cid * RCH
            pltpu.sync_copy(acc.at[pl.ds(base_r, RCH)], fbuf_v)

            @pl.when(c == 0)
            def _():
                pltpu.sync_copy(fbuf_v, out0.at[pl.ds(base_r, RCH)])

            @pl.when(c == 1)
            def _():
                pltpu.sync_copy(fbuf_v, out1.at[pl.ds(base_r, RCH)])
        return 0
    lax.fori_loop(0, RITER, out_chunk, 0)


def _add_body(a_ref, b_ref, o_ref):
    o_ref[...] = a_ref[...] + b_ref[...]


@jax.jit
def _run(h, src, dst, e):
    mesh = plsc.VectorSubcoreMesh(core_axis_name="c", subcore_axis_name="s",
                                  num_cores=NC, num_subcores=NS)
    f = pl.kernel(
        _body,
        out_type=(jax.ShapeDtypeStruct((N, D), jnp.float32),
                  jax.ShapeDtypeStruct((N, D), jnp.float32)),
        mesh=mesh,
        scratch_types=[
            pltpu.VMEM((UN, CH), jnp.int32),
            pltpu.VMEM((UN, CH), jnp.int32),
            pltpu.VMEM((UN, CH), jnp.float32),
            pltpu.VMEM((2, CH, D), jnp.float32),
            pltpu.VMEM((2, CH, D), jnp.float32),
            pltpu.VMEM((RCH,), jnp.int32),
            pltpu.VMEM((RCH, D), jnp.float32),
            pltpu.VMEM_SHARED((N, D), jnp.float32),
            pltpu.SemaphoreType.DMA((2,)),
            pltpu.SemaphoreType.DMA((2,)),
            pltpu.SemaphoreType.DMA((UN,)),
        ],
        compiler_params=pltpu.CompilerParams(needs_layout_passes=False,
                                             use_tc_tiling_on_sc=False),
    )
    o0, o1 = f(h, src, dst, e)
    add = pl.pallas_call(
        _add_body,
        out_shape=jax.ShapeDtypeStruct((N, D), jnp.float32),
        grid=(10,),
        in_specs=[pl.BlockSpec((N // 10, D), lambda i: (i, 0)),
                  pl.BlockSpec((N // 10, D), lambda i: (i, 0))],
        out_specs=pl.BlockSpec((N // 10, D), lambda i: (i, 0)),
    )
    return add(o0, o1)


def kernel(t, x, edge_index):
    h = x[: N * D].reshape(N, D)
    e = x[N * D:]
    src = edge_index[0].astype(jnp.int32)
    dst = edge_index[1].astype(jnp.int32)
    h_new = _run(h, src, dst, e)
    return jnp.concatenate([h_new.reshape(-1), jnp.zeros((E,), x.dtype)])
